# baseline (device time: 205026 ns/iter reference)
import jax
import jax.numpy as jnp
import numpy as np
from jax import lax
from jax.experimental import pallas as pl
from jax.experimental.pallas import tpu as pltpu

N_DEV = 32

_RING = np.array(
    [0, 1, 2, 5, 6, 7, 4, 3,
     11, 12, 15, 14, 13, 10, 9,
     17, 18, 21, 22, 23, 20, 19,
     27, 28, 31, 30, 29, 26, 25, 24,
     16, 8],
    dtype=np.int32,
)
_INV = np.argsort(_RING).astype(np.int32)

N_FWD = 16
N_BWD = 15
N_SC_SEMS = 4
N_PIECES = 4


def kernel(x, w_mat):
    m_per, k = x.shape
    n = w_mat.shape[1]
    m_tot = N_DEV * m_per

    def body(ring_ref, inv_ref, x_ref, w_ref, out_ref, gx_ref, amax_ref,
             ring_send_sems, ring_recv_sems, sc_send_sems, sc_recv_sem):
        my = lax.axis_index("i")
        r = inv_ref[my]
        left = ring_ref[jnp.mod(r - 1, N_DEV)]
        right = ring_ref[jnp.mod(r + 1, N_DEV)]

        barrier_sem = pltpu.get_barrier_semaphore()
        pl.semaphore_signal(barrier_sem, inc=1, device_id=(left,),
                            device_id_type=pl.DeviceIdType.MESH)
        pl.semaphore_signal(barrier_sem, inc=1, device_id=(right,),
                            device_id_type=pl.DeviceIdType.MESH)
        pl.semaphore_wait(barrier_sem, 2)

        hm = m_per // N_PIECES

        def mk_send(origin, half, tgt, dir_idx):
            return pltpu.make_async_remote_copy(
                src_ref=gx_ref.at[origin, pl.ds(half * hm, hm)],
                dst_ref=gx_ref.at[origin, pl.ds(half * hm, hm)],
                send_sem=ring_send_sems.at[dir_idx, half],
                recv_sem=ring_recv_sems.at[origin, half],
                device_id=(tgt,),
                device_id_type=pl.DeviceIdType.MESH,
            )

        def wait_half(origin, half):
            pltpu.make_async_remote_copy(
                src_ref=gx_ref.at[origin, pl.ds(half * hm, hm)],
                dst_ref=gx_ref.at[origin, pl.ds(half * hm, hm)],
                send_sem=ring_send_sems.at[0, half],
                recv_sem=ring_recv_sems.at[origin, half],
                device_id=(my,),
                device_id_type=pl.DeviceIdType.MESH,
            ).wait_recv()

        xb0 = x_ref[...].astype(jnp.bfloat16)
        gx_ref[pl.ds(my, 1)] = xb0.reshape(1, m_per, k)

        fwd = [mk_send(my, p, right, 0) for p in range(N_PIECES)]
        bwd = [mk_send(my, p, left, 1) for p in range(N_PIECES)]
        for d in (*fwd, *bwd):
            d.start()

        w = w_ref[...].astype(jnp.bfloat16)
        y0 = jnp.maximum(
            jnp.dot(xb0, w, preferred_element_type=jnp.float32), 0.0)
        out_ref[pl.ds(my * m_per, m_per), :] = y0
        amax = jnp.max(y0)

        def gemm_chunk(origin, amax):
            xb = gx_ref[pl.ds(origin, 1)].reshape(m_per, k)
            yb = jnp.maximum(
                jnp.dot(xb, w, preferred_element_type=jnp.float32), 0.0)
            out_ref[pl.ds(origin * m_per, m_per), :] = yb
            return jnp.maximum(amax, jnp.max(yb))

        for h in range(N_FWD):
            rf = ring_ref[jnp.mod(r - h - 1, N_DEV)]
            rb = None
            if h < N_BWD:
                rb = ring_ref[jnp.mod(r + h + 1, N_DEV)]
            for half in range(N_PIECES):
                wait_half(rf, half)
                if h < N_FWD - 1:
                    fwd[half].wait_send()
                    fwd[half] = mk_send(rf, half, right, 0)
                    fwd[half].start()
                if rb is not None:
                    wait_half(rb, half)
                    if h < N_BWD - 1:
                        bwd[half].wait_send()
                        bwd[half] = mk_send(rb, half, left, 1)
                        bwd[half].start()
            amax = gemm_chunk(rf, amax)
            if rb is not None:
                amax = gemm_chunk(rb, amax)
        for d in (*fwd, *bwd):
            d.wait_send()

        amax_ref[pl.ds(my, 1)] = jnp.full((1, 128), amax, jnp.float32)
        descs = []
        for d in range(1, N_DEV):
            tgt = jnp.mod(my + d, N_DEV)
            s = pltpu.make_async_remote_copy(
                src_ref=amax_ref.at[my],
                dst_ref=amax_ref.at[my],
                send_sem=sc_send_sems.at[(d - 1) % N_SC_SEMS],
                recv_sem=sc_recv_sem,
                device_id=(tgt,),
                device_id_type=pl.DeviceIdType.MESH,
            )
            if d - 1 >= N_SC_SEMS:
                descs[d - 1 - N_SC_SEMS].wait_send()
            s.start()
            descs.append(s)
        for i in range(N_DEV - 1 - N_SC_SEMS, N_DEV - 1):
            descs[i].wait_send()
        for d in range(1, N_DEV):
            src = jnp.mod(my + d, N_DEV)
            pltpu.make_async_remote_copy(
                src_ref=amax_ref.at[src],
                dst_ref=amax_ref.at[src],
                send_sem=sc_send_sems.at[0],
                recv_sem=sc_recv_sem,
                device_id=(my,),
                device_id_type=pl.DeviceIdType.MESH,
            ).wait_recv()

        amax_g = jnp.max(amax_ref[...])
        scale = jnp.maximum(amax_g, 1e-30) / 448.0

        y = out_ref[...]
        q = (y / scale).astype(jnp.float8_e4m3fn).astype(jnp.float32)
        out_ref[...] = q * scale

    ring = jnp.asarray(_RING)
    inv = jnp.asarray(_INV)

    return pl.pallas_call(
        body,
        out_shape=jax.ShapeDtypeStruct((m_tot, n), jnp.float32),
        in_specs=[
            pl.BlockSpec(memory_space=pltpu.SMEM),
            pl.BlockSpec(memory_space=pltpu.SMEM),
            pl.BlockSpec(memory_space=pltpu.VMEM),
            pl.BlockSpec(memory_space=pltpu.VMEM),
        ],
        out_specs=pl.BlockSpec(memory_space=pltpu.VMEM),
        scratch_shapes=[
            pltpu.VMEM((N_DEV, m_per, k), jnp.bfloat16),
            pltpu.VMEM((N_DEV, 128), jnp.float32),
            pltpu.SemaphoreType.DMA((2, N_PIECES)),
            pltpu.SemaphoreType.DMA((N_DEV, N_PIECES)),
            pltpu.SemaphoreType.DMA((N_SC_SEMS,)),
            pltpu.SemaphoreType.DMA,
        ],
        compiler_params=pltpu.CompilerParams(
            collective_id=0,
            vmem_limit_bytes=100 * 1024 * 1024,
        ),
    )(ring, inv, x, w_mat)


# device time: 198482 ns/iter; 1.0330x vs baseline; 1.0330x over previous
import jax
import jax.numpy as jnp
import numpy as np
from jax import lax
from jax.experimental import pallas as pl
from jax.experimental.pallas import tpu as pltpu

N_DEV = 32

_RING = np.array(
    [0, 1, 2, 5, 6, 7, 4, 3,
     11, 12, 15, 14, 13, 10, 9,
     17, 18, 21, 22, 23, 20, 19,
     27, 28, 31, 30, 29, 26, 25, 24,
     16, 8],
    dtype=np.int32,
)
_INV = np.argsort(_RING).astype(np.int32)

N_PIECES = 4
FWD_PIECES = (0, 1)
BWD_PIECES = (2, 3)


def kernel(x, w_mat):
    m_per, k = x.shape
    n = w_mat.shape[1]
    m_tot = N_DEV * m_per

    def body(ring_ref, inv_ref, x_ref, w_ref, out_ref, gx_ref, amax_ref,
             ring_send_sems, ring_recv_sems, sc_send_sems, sc_recv_sem):
        my = lax.axis_index("i")
        r = inv_ref[my]
        left = ring_ref[jnp.mod(r - 1, N_DEV)]
        right = ring_ref[jnp.mod(r + 1, N_DEV)]

        barrier_sem = pltpu.get_barrier_semaphore()
        pl.semaphore_signal(barrier_sem, inc=1, device_id=(left,),
                            device_id_type=pl.DeviceIdType.MESH)
        pl.semaphore_signal(barrier_sem, inc=1, device_id=(right,),
                            device_id_type=pl.DeviceIdType.MESH)
        pl.semaphore_wait(barrier_sem, 2)

        hm = m_per // N_PIECES

        def mk_send(origin, half, tgt, dir_idx):
            return pltpu.make_async_remote_copy(
                src_ref=gx_ref.at[origin, pl.ds(half * hm, hm)],
                dst_ref=gx_ref.at[origin, pl.ds(half * hm, hm)],
                send_sem=ring_send_sems.at[dir_idx, half],
                recv_sem=ring_recv_sems.at[origin, half],
                device_id=(tgt,),
                device_id_type=pl.DeviceIdType.MESH,
            )

        def wait_half(origin, half):
            pltpu.make_async_remote_copy(
                src_ref=gx_ref.at[origin, pl.ds(half * hm, hm)],
                dst_ref=gx_ref.at[origin, pl.ds(half * hm, hm)],
                send_sem=ring_send_sems.at[0, half],
                recv_sem=ring_recv_sems.at[origin, half],
                device_id=(my,),
                device_id_type=pl.DeviceIdType.MESH,
            ).wait_recv()

        fwd = [None] * N_PIECES
        bwd = [None] * N_PIECES
        for p in range(N_PIECES):
            gx_ref[pl.ds(my, 1), pl.ds(p * hm, hm)] = (
                x_ref[pl.ds(p * hm, hm), :].astype(jnp.bfloat16)
                .reshape(1, hm, k))
            fwd[p] = mk_send(my, p, right, 0)
            fwd[p].start()
            bwd[p] = mk_send(my, p, left, 1)
            bwd[p].start()

        w = w_ref[...].astype(jnp.bfloat16)
        xb0 = gx_ref[pl.ds(my, 1)].reshape(m_per, k)
        y0 = jnp.maximum(
            jnp.dot(xb0, w, preferred_element_type=jnp.float32), 0.0)
        out_ref[pl.ds(my * m_per, m_per), :] = y0
        amax = jnp.max(y0)

        def gemm_chunk(origin, amax):
            xb = gx_ref[pl.ds(origin, 1)].reshape(m_per, k)
            yb = jnp.maximum(
                jnp.dot(xb, w, preferred_element_type=jnp.float32), 0.0)
            out_ref[pl.ds(origin * m_per, m_per), :] = yb
            return jnp.maximum(amax, jnp.max(yb))

        for h in range(16):
            rf = ring_ref[jnp.mod(r - h - 1, N_DEV)]
            rb = ring_ref[jnp.mod(r + h + 1, N_DEV)]
            f_recv = range(N_PIECES) if h <= 14 else FWD_PIECES
            b_recv = range(N_PIECES) if h <= 14 else BWD_PIECES
            f_fw = range(N_PIECES) if h <= 13 else (FWD_PIECES if h == 14 else ())
            b_fw = range(N_PIECES) if h <= 13 else (BWD_PIECES if h == 14 else ())
            for p in range(N_PIECES):
                if p in f_recv:
                    wait_half(rf, p)
                    if p in f_fw:
                        fwd[p].wait_send()
                        fwd[p] = mk_send(rf, p, right, 0)
                        fwd[p].start()
                if p in b_recv:
                    wait_half(rb, p)
                    if p in b_fw:
                        bwd[p].wait_send()
                        bwd[p] = mk_send(rb, p, left, 1)
                        bwd[p].start()
            amax = gemm_chunk(rf, amax)
            if h <= 14:
                amax = gemm_chunk(rb, amax)
        for d in (*fwd, *bwd):
            d.wait_send()

        amax_ref[pl.ds(my, 1)] = jnp.full((1, 128), amax, jnp.float32)
        descs = []
        for d in range(1, N_DEV):
            tgt = jnp.mod(my + d, N_DEV)
            s = pltpu.make_async_remote_copy(
                src_ref=amax_ref.at[my],
                dst_ref=amax_ref.at[my],
                send_sem=sc_send_sems.at[d - 1],
                recv_sem=sc_recv_sem,
                device_id=(tgt,),
                device_id_type=pl.DeviceIdType.MESH,
            )
            s.start()
            descs.append(s)
        for d in range(1, N_DEV):
            src = jnp.mod(my + d, N_DEV)
            pltpu.make_async_remote_copy(
                src_ref=amax_ref.at[src],
                dst_ref=amax_ref.at[src],
                send_sem=sc_send_sems.at[0],
                recv_sem=sc_recv_sem,
                device_id=(my,),
                device_id_type=pl.DeviceIdType.MESH,
            ).wait_recv()

        amax_g = jnp.max(amax_ref[...])
        scale = jnp.maximum(amax_g, 1e-30) / 448.0

        y = out_ref[...]
        q = (y / scale).astype(jnp.float8_e4m3fn).astype(jnp.float32)
        out_ref[...] = q * scale

        for s in descs:
            s.wait_send()

    ring = jnp.asarray(_RING)
    inv = jnp.asarray(_INV)

    return pl.pallas_call(
        body,
        out_shape=jax.ShapeDtypeStruct((m_tot, n), jnp.float32),
        in_specs=[
            pl.BlockSpec(memory_space=pltpu.SMEM),
            pl.BlockSpec(memory_space=pltpu.SMEM),
            pl.BlockSpec(memory_space=pltpu.VMEM),
            pl.BlockSpec(memory_space=pltpu.VMEM),
        ],
        out_specs=pl.BlockSpec(memory_space=pltpu.VMEM),
        scratch_shapes=[
            pltpu.VMEM((N_DEV, m_per, k), jnp.bfloat16),
            pltpu.VMEM((N_DEV, 128), jnp.float32),
            pltpu.SemaphoreType.DMA((2, N_PIECES)),
            pltpu.SemaphoreType.DMA((N_DEV, N_PIECES)),
            pltpu.SemaphoreType.DMA((N_DEV - 1,)),
            pltpu.SemaphoreType.DMA,
        ],
        compiler_params=pltpu.CompilerParams(
            collective_id=0,
            vmem_limit_bytes=100 * 1024 * 1024,
        ),
    )(ring, inv, x, w_mat)


# device time: 196934 ns/iter; 1.0411x vs baseline; 1.0079x over previous
import jax
import jax.numpy as jnp
import numpy as np
from jax import lax
from jax.experimental import pallas as pl
from jax.experimental.pallas import tpu as pltpu

N_DEV = 32

_RING = np.array(
    [0, 1, 2, 5, 6, 7, 4, 3,
     11, 12, 15, 14, 13, 10, 9,
     17, 18, 21, 22, 23, 20, 19,
     27, 28, 31, 30, 29, 26, 25, 24,
     16, 8],
    dtype=np.int32,
)
_INV = np.argsort(_RING).astype(np.int32)

N_PIECES = 4
FWD_PIECES = (0, 1)
BWD_PIECES = (2, 3)


def kernel(x, w_mat):
    m_per, k = x.shape
    n = w_mat.shape[1]
    m_tot = N_DEV * m_per

    def body(ring_ref, inv_ref, x_ref, w_ref, out_ref, gx_ref, y_ref,
             amax_ref, ring_send_sems, ring_recv_sems, sc_send_sems,
             sc_recv_sem):
        my = lax.axis_index("i")
        r = inv_ref[my]
        left = ring_ref[jnp.mod(r - 1, N_DEV)]
        right = ring_ref[jnp.mod(r + 1, N_DEV)]

        barrier_sem = pltpu.get_barrier_semaphore()
        pl.semaphore_signal(barrier_sem, inc=1, device_id=(left,),
                            device_id_type=pl.DeviceIdType.MESH)
        pl.semaphore_signal(barrier_sem, inc=1, device_id=(right,),
                            device_id_type=pl.DeviceIdType.MESH)
        pl.semaphore_wait(barrier_sem, 2)

        hm = m_per // N_PIECES

        def mk_send(origin, half, tgt, dir_idx):
            return pltpu.make_async_remote_copy(
                src_ref=gx_ref.at[origin, pl.ds(half * hm, hm)],
                dst_ref=gx_ref.at[origin, pl.ds(half * hm, hm)],
                send_sem=ring_send_sems.at[dir_idx, half],
                recv_sem=ring_recv_sems.at[origin, half],
                device_id=(tgt,),
                device_id_type=pl.DeviceIdType.MESH,
            )

        def wait_half(origin, half):
            pltpu.make_async_remote_copy(
                src_ref=gx_ref.at[origin, pl.ds(half * hm, hm)],
                dst_ref=gx_ref.at[origin, pl.ds(half * hm, hm)],
                send_sem=ring_send_sems.at[0, half],
                recv_sem=ring_recv_sems.at[origin, half],
                device_id=(my,),
                device_id_type=pl.DeviceIdType.MESH,
            ).wait_recv()

        fwd = [None] * N_PIECES
        bwd = [None] * N_PIECES
        for p in range(N_PIECES):
            gx_ref[pl.ds(my, 1), pl.ds(p * hm, hm)] = (
                x_ref[pl.ds(p * hm, hm), :].astype(jnp.bfloat16)
                .reshape(1, hm, k))
            fwd[p] = mk_send(my, p, right, 0)
            fwd[p].start()
            bwd[p] = mk_send(my, p, left, 1)
            bwd[p].start()

        w = w_ref[...].astype(jnp.bfloat16)
        xb0 = gx_ref[pl.ds(my, 1)].reshape(m_per, k)
        y0 = jnp.maximum(
            jnp.dot(xb0, w, preferred_element_type=jnp.float32), 0.0)
        y_ref[pl.ds(my * m_per, m_per), :] = y0
        amax = jnp.max(y0)

        def gemm_chunk(origin, amax):
            xb = gx_ref[pl.ds(origin, 1)].reshape(m_per, k)
            yb = jnp.maximum(
                jnp.dot(xb, w, preferred_element_type=jnp.float32), 0.0)
            y_ref[pl.ds(origin * m_per, m_per), :] = yb
            return jnp.maximum(amax, jnp.max(yb))

        for h in range(16):
            rf = ring_ref[jnp.mod(r - h - 1, N_DEV)]
            rb = ring_ref[jnp.mod(r + h + 1, N_DEV)]
            f_recv = range(N_PIECES) if h <= 14 else FWD_PIECES
            b_recv = range(N_PIECES) if h <= 14 else BWD_PIECES
            f_fw = range(N_PIECES) if h <= 13 else (FWD_PIECES if h == 14 else ())
            b_fw = range(N_PIECES) if h <= 13 else (BWD_PIECES if h == 14 else ())
            for p in range(N_PIECES):
                if p in f_recv:
                    wait_half(rf, p)
                    if p in f_fw:
                        fwd[p].wait_send()
                        fwd[p] = mk_send(rf, p, right, 0)
                        fwd[p].start()
                if p in b_recv:
                    wait_half(rb, p)
                    if p in b_fw:
                        bwd[p].wait_send()
                        bwd[p] = mk_send(rb, p, left, 1)
                        bwd[p].start()
            amax = gemm_chunk(rf, amax)
            if h <= 14:
                amax = gemm_chunk(rb, amax)
        for d in (*fwd, *bwd):
            d.wait_send()

        amax_ref[pl.ds(my, 1)] = jnp.full((1, 128), amax, jnp.float32)
        descs = []
        for d in range(1, N_DEV):
            tgt = jnp.mod(my + d, N_DEV)
            s = pltpu.make_async_remote_copy(
                src_ref=amax_ref.at[my],
                dst_ref=amax_ref.at[my],
                send_sem=sc_send_sems.at[d - 1],
                recv_sem=sc_recv_sem,
                device_id=(tgt,),
                device_id_type=pl.DeviceIdType.MESH,
            )
            s.start()
            descs.append(s)
        for d in range(1, N_DEV):
            src = jnp.mod(my + d, N_DEV)
            pltpu.make_async_remote_copy(
                src_ref=amax_ref.at[src],
                dst_ref=amax_ref.at[src],
                send_sem=sc_send_sems.at[0],
                recv_sem=sc_recv_sem,
                device_id=(my,),
                device_id_type=pl.DeviceIdType.MESH,
            ).wait_recv()

        amax_g = jnp.maximum(jnp.max(amax_ref[...]), 1e-30)
        scale = amax_g / 448.0
        inv_scale = 448.0 / amax_g

        y = y_ref[...]
        q = (y * inv_scale).astype(jnp.float8_e4m3fn).astype(jnp.float32)
        out_ref[...] = (q * scale).astype(jnp.bfloat16)

        for s in descs:
            s.wait_send()

    ring = jnp.asarray(_RING)
    inv = jnp.asarray(_INV)

    return pl.pallas_call(
        body,
        out_shape=jax.ShapeDtypeStruct((m_tot, n), jnp.bfloat16),
        in_specs=[
            pl.BlockSpec(memory_space=pltpu.SMEM),
            pl.BlockSpec(memory_space=pltpu.SMEM),
            pl.BlockSpec(memory_space=pltpu.VMEM),
            pl.BlockSpec(memory_space=pltpu.VMEM),
        ],
        out_specs=pl.BlockSpec(memory_space=pltpu.VMEM),
        scratch_shapes=[
            pltpu.VMEM((N_DEV, m_per, k), jnp.bfloat16),
            pltpu.VMEM((m_tot, n), jnp.float32),
            pltpu.VMEM((N_DEV, 128), jnp.float32),
            pltpu.SemaphoreType.DMA((2, N_PIECES)),
            pltpu.SemaphoreType.DMA((N_DEV, N_PIECES)),
            pltpu.SemaphoreType.DMA((N_DEV - 1,)),
            pltpu.SemaphoreType.DMA,
        ],
        compiler_params=pltpu.CompilerParams(
            collective_id=0,
            vmem_limit_bytes=100 * 1024 * 1024,
        ),
    )(ring, inv, x, w_mat)
